# baseline (speedup 1.0000x reference)
import functools

import jax
import jax.numpy as jnp
import numpy as np
from jax import lax
from jax.experimental import pallas as pl
from jax.experimental.pallas import tpu as pltpu
from jax.experimental.pallas import tpu_sc as plsc

D = 256
NH = 8
NL = 4
NP = 4
DH = D // NH
SPAT = [(64, 64), (32, 32), (16, 16), (8, 8)]
LQ = sum(h * w for h, w in SPAT)
NIMG = 4
M = NIMG * LQ
TAB = NIMG * NH * LQ
R = 1360
NBLK = M // R
BLK_PER_IMG = LQ // R
NWORK = 32
PER_W = M // NWORK

_LVL_START = np.array([0, 4096, 5120, 5376], dtype=np.int64)

_wl_lane = np.tile(np.repeat(np.array([w for _, w in SPAT], np.float32), NP), NH)
_hl_lane = np.tile(np.repeat(np.array([h for h, _ in SPAT], np.float32), NP), NH)
_base_lane = (np.repeat(np.arange(NH, dtype=np.int64) * LQ, NL * NP)
              + np.tile(np.repeat(_LVL_START, NP), NH)).astype(np.int32)
_wli_lane = _wl_lane.astype(np.int32)

_LCF = np.zeros((8, 128), np.float32)
_LCF[0] = _wl_lane
_LCF[1] = _hl_lane
_LCI = np.zeros((8, 128), np.int32)
_LCI[0] = _base_lane
_LCI[1] = _wli_lane
_G = np.kron(np.eye(NH, dtype=np.float32), np.ones((16, 16), np.float32))


def _stage_a_body(q_ref, rpx_ref, rpy_ref, wv_ref, bv_ref, wox_ref, box_ref,
                  woy_ref, boy_ref, wat_ref, bat_ref, g_ref, lcf_ref, lci_ref,
                  vbf_ref, idx_ref, w_ref):
    pid = pl.program_id(0)
    nbase = (pid // BLK_PER_IMG) * (NH * LQ)
    q = q_ref[...]
    v = jnp.dot(q, wv_ref[...], preferred_element_type=jnp.float32) + bv_ref[0:1, :]
    vbf_ref[...] = v.astype(jnp.bfloat16)
    sx = jnp.dot(v, wox_ref[...], preferred_element_type=jnp.float32) + box_ref[0:1, :]
    sy = jnp.dot(v, woy_ref[...], preferred_element_type=jnp.float32) + boy_ref[0:1, :]
    al = jnp.dot(v, wat_ref[...], preferred_element_type=jnp.float32) + bat_ref[0:1, :]
    al = al - jnp.max(al, axis=1, keepdims=True)
    e = jnp.exp(jnp.maximum(al, -75.0))
    s = jnp.dot(e, g_ref[...], preferred_element_type=jnp.float32)
    aw = e / s
    wlf = lcf_ref[0:1, :]
    hlf = lcf_ref[1:2, :]
    base = lci_ref[0:1, :] + nbase
    wli = lci_ref[1:2, :]
    x = rpx_ref[...] * wlf + sx - 0.5
    y = rpy_ref[...] * hlf + sy - 0.5
    x0 = jnp.floor(x)
    y0 = jnp.floor(y)
    fx1 = x - x0
    fx0 = 1.0 - fx1
    fy1 = y - y0
    fy0 = 1.0 - fy1
    idx_parts = []
    w_parts = []
    for dy in (0, 1):
        yi = y0 + dy
        vy = (yi >= 0.0) & (yi <= hlf - 1.0)
        yc = jnp.clip(yi, 0.0, hlf - 1.0).astype(jnp.int32)
        wy = fy1 if dy else fy0
        for dx in (0, 1):
            xi = x0 + dx
            vx = (xi >= 0.0) & (xi <= wlf - 1.0)
            xc = jnp.clip(xi, 0.0, wlf - 1.0).astype(jnp.int32)
            wx = fx1 if dx else fx0
            wc = aw * wy * wx * (vy & vx).astype(jnp.float32)
            idx_parts.append(base + yc * wli + xc)
            w_parts.append(wc)
    idx_ref[...] = jnp.concatenate(idx_parts, axis=1)
    w_ref[...] = jnp.concatenate(w_parts, axis=1)


def _row_spec(cols):
    return pl.BlockSpec((R, cols), lambda i: (i, 0))


def _full_spec(rows, cols):
    return pl.BlockSpec((rows, cols), lambda i: (0, 0))


_A_GRID = (NBLK,)
_A_IN_SPECS = [
    _row_spec(D),
    _row_spec(128),
    _row_spec(128),
    _full_spec(D, D),
    _full_spec(8, D),
    _full_spec(D, 128),
    _full_spec(8, 128),
    _full_spec(D, 128),
    _full_spec(8, 128),
    _full_spec(D, 128),
    _full_spec(8, 128),
    _full_spec(128, 128),
    _full_spec(8, 128),
    _full_spec(8, 128),
]
_A_OUT_SPECS = [_row_spec(D), _row_spec(512), _row_spec(512)]
_A_OUT_SHAPE = [
    jax.ShapeDtypeStruct((M, D), jnp.bfloat16),
    jax.ShapeDtypeStruct((M, 512), jnp.int32),
    jax.ShapeDtypeStruct((M, 512), jnp.float32),
]

_PERM = np.concatenate([
    h * DH + np.concatenate([np.arange(0, DH, 2), np.arange(1, DH, 2)])
    for h in range(NH)
])


def _mm_body(x_ref, w_ref, b_ref, o_ref):
    o_ref[...] = (jnp.dot(x_ref[...], w_ref[...],
                          preferred_element_type=jnp.float32) + b_ref[0:1, :])


_MM_IN_SPECS = [_row_spec(D), _full_spec(D, D), _full_spec(8, D)]
_MM_OUT_SPEC = _row_spec(D)


CH = 2
NCHK = PER_W // CH


def _sc_sample_body(vt_hbm, idx_hbm, w_hbm, out_hbm,
                    idxv0, idxv1, wv0, wv1, rows0, rows1, outv0, outv1,
                    semi0, semi1, semg0, semg1, semo0, semo1):
    wid = lax.axis_index("s") * 2 + lax.axis_index("c")
    m0 = wid * PER_W
    idxv = (idxv0, idxv1)
    wv = (wv0, wv1)
    rows = (rows0, rows1)
    outv = (outv0, outv1)
    semi = (semi0, semi1)
    semg = (semg0, semg1)
    semo = (semo0, semo1)

    def load_and_gather(g, b):
        mg = m0 + g * CH
        ci = pltpu.async_copy(idx_hbm.at[pl.ds(mg, CH)], idxv[b], semi[b])
        cw = pltpu.async_copy(w_hbm.at[pl.ds(mg, CH)], wv[b], semi[b])
        ci.wait()
        cw.wait()
        for mm in range(CH):
            for c in range(4):
                pltpu.async_copy(vt_hbm.at[idxv[b].at[mm, c]],
                                 rows[b].at[mm, c], semg[b])

    def wait_gathers(b):
        for mm in range(CH):
            for c in range(4):
                pltpu.make_async_copy(vt_hbm.at[idxv[b].at[mm, c]],
                                      rows[b].at[mm, c], semg[b]).wait()

    def compute(g, b):
        for mm in range(CH):
            def body_h(h, c2, mm=mm):
                ae = [jnp.zeros((16,), jnp.float32) for _ in range(4)]
                ao = [jnp.zeros((16,), jnp.float32) for _ in range(4)]
                for c in range(4):
                    wrow = wv[b][mm, c, pl.ds(h * 16, 16)]
                    for j in range(16):
                        pos = h * 16 + j
                        wvec = jnp.full((16,), wrow[j], jnp.float32)
                        u = rows[b][mm, c, pos, :]
                        lo = lax.bitcast_convert_type(u << 16, jnp.float32)
                        hi = lax.bitcast_convert_type(u & jnp.int32(-65536),
                                                      jnp.float32)
                        ae[c] = ae[c] + lo * wvec
                        ao[c] = ao[c] + hi * wvec
                outv[b][mm, h, pl.ds(0, 16)] = (ae[0] + ae[1]) + (ae[2] + ae[3])
                outv[b][mm, h, pl.ds(16, 16)] = (ao[0] + ao[1]) + (ao[2] + ao[3])
                return c2

            lax.fori_loop(0, NH, body_h, 0)
        mg = m0 + g * CH
        pltpu.async_copy(outv[b], out_hbm.at[pl.ds(mg, CH)], semo[b])

    def wait_out(g, b):
        mg = m0 + g * CH
        pltpu.make_async_copy(outv[b], out_hbm.at[pl.ds(mg, CH)],
                              semo[b]).wait()

    load_and_gather(0, 0)

    def outer(go, carry):
        for b in (0, 1):
            g = go * 2 + b
            q = 1 - b

            @pl.when(g + 1 < NCHK)
            def _():
                load_and_gather(g + 1, q)

            wait_gathers(b)

            @pl.when(g >= 2)
            def _():
                wait_out(g - 2, b)

            compute(g, b)
        return carry

    lax.fori_loop(0, NCHK // 2, outer, 0)
    wait_out(NCHK - 2, 0)
    wait_out(NCHK - 1, 1)


@functools.lru_cache(maxsize=1)
def _get_sc_kernel():
    mesh = plsc.VectorSubcoreMesh(core_axis_name="c", subcore_axis_name="s")
    return pl.kernel(
        _sc_sample_body,
        out_type=jax.ShapeDtypeStruct((M, NH, DH), jnp.float32),
        mesh=mesh,
        compiler_params=pltpu.CompilerParams(use_tc_tiling_on_sc=False),
        scratch_types=[
            pltpu.VMEM((CH, 4, 128), jnp.int32),
            pltpu.VMEM((CH, 4, 128), jnp.int32),
            pltpu.VMEM((CH, 4, 128), jnp.float32),
            pltpu.VMEM((CH, 4, 128), jnp.float32),
            pltpu.VMEM((CH, 4, 128, DH // 2), jnp.int32),
            pltpu.VMEM((CH, 4, 128, DH // 2), jnp.int32),
            pltpu.VMEM((CH, NH, DH), jnp.float32),
            pltpu.VMEM((CH, NH, DH), jnp.float32),
            pltpu.SemaphoreType.DMA,
            pltpu.SemaphoreType.DMA,
            pltpu.SemaphoreType.DMA,
            pltpu.SemaphoreType.DMA,
            pltpu.SemaphoreType.DMA,
            pltpu.SemaphoreType.DMA,
        ],
    )


def _stage_a(q2, rpx, rpy, W_value, b_value, W_off, b_off, W_attn, b_attn):
    wv_t = W_value.T
    wox_t = W_off[0::2].T
    woy_t = W_off[1::2].T
    wat_t = W_attn.T
    bv8 = jnp.broadcast_to(b_value[None, :], (8, D))
    box8 = jnp.broadcast_to(b_off[0::2][None, :], (8, 128))
    boy8 = jnp.broadcast_to(b_off[1::2][None, :], (8, 128))
    bat8 = jnp.broadcast_to(b_attn[None, :], (8, 128))
    return pl.pallas_call(
        _stage_a_body,
        grid=_A_GRID,
        in_specs=_A_IN_SPECS,
        out_specs=_A_OUT_SPECS,
        out_shape=_A_OUT_SHAPE,
    )(q2, rpx, rpy, wv_t, bv8, wox_t, box8, woy_t, boy8, wat_t, bat8,
      jnp.asarray(_G), jnp.asarray(_LCF), jnp.asarray(_LCI))


def kernel(query, reference_points, input_spatial_shapes,
           input_level_start_index, temporal_points, temporal_points_weights,
           W_value, b_value, W_off, b_off, W_attn, b_attn, W_out, b_out):
    q2 = query.reshape(M, D)
    rp = reference_points.reshape(M, NL, 2)
    rpx = jnp.broadcast_to(rp[:, None, :, None, 0], (M, NH, NL, NP)).reshape(M, 128)
    rpy = jnp.broadcast_to(rp[:, None, :, None, 1], (M, NH, NL, NP)).reshape(M, 128)
    vbf, idx2, w2 = _stage_a(q2, rpx, rpy, W_value, b_value, W_off, b_off,
                             W_attn, b_attn)
    vt = vbf.reshape(NIMG, LQ, NH, DH).transpose(0, 2, 1, 3).reshape(TAB, DH)
    vt32 = lax.bitcast_convert_type(vt.reshape(TAB, DH // 2, 2), jnp.int32)
    sampled = _get_sc_kernel()(vt32, idx2.reshape(M, 4, 128),
                               w2.reshape(M, 4, 128))
    out2 = pl.pallas_call(
        _mm_body,
        grid=_A_GRID,
        in_specs=_MM_IN_SPECS,
        out_specs=_MM_OUT_SPEC,
        out_shape=jax.ShapeDtypeStruct((M, D), jnp.float32),
    )(sampled.reshape(M, D), W_out.T[_PERM],
      jnp.broadcast_to(b_out[None, :], (8, D)))
    return out2.reshape(NIMG, LQ, D)


# speedup vs baseline: 1.0056x; 1.0056x over previous
import functools

import jax
import jax.numpy as jnp
import numpy as np
from jax import lax
from jax.experimental import pallas as pl
from jax.experimental.pallas import tpu as pltpu
from jax.experimental.pallas import tpu_sc as plsc

D = 256
NH = 8
NL = 4
NP = 4
DH = D // NH
SPAT = [(64, 64), (32, 32), (16, 16), (8, 8)]
LQ = sum(h * w for h, w in SPAT)
NIMG = 4
M = NIMG * LQ
TAB = NIMG * NH * LQ
R = 1360
NBLK = M // R
BLK_PER_IMG = LQ // R
NWORK = 32
PER_W = M // NWORK

_LVL_START = np.array([0, 4096, 5120, 5376], dtype=np.int64)

_wl_lane = np.tile(np.repeat(np.array([w for _, w in SPAT], np.float32), NP), NH)
_hl_lane = np.tile(np.repeat(np.array([h for h, _ in SPAT], np.float32), NP), NH)
_base_lane = (np.repeat(np.arange(NH, dtype=np.int64) * LQ, NL * NP)
              + np.tile(np.repeat(_LVL_START, NP), NH)).astype(np.int32)
_wli_lane = _wl_lane.astype(np.int32)

_LCF = np.zeros((8, 128), np.float32)
_LCF[0] = _wl_lane
_LCF[1] = _hl_lane
_LCI = np.zeros((8, 128), np.int32)
_LCI[0] = _base_lane
_LCI[1] = _wli_lane
_G = np.kron(np.eye(NH, dtype=np.float32), np.ones((16, 16), np.float32))


def _stage_a_body(q_ref, rpx_ref, rpy_ref, wv_ref, bv_ref, wox_ref, box_ref,
                  woy_ref, boy_ref, wat_ref, bat_ref, g_ref, lcf_ref, lci_ref,
                  vbf_ref, idx_ref, w_ref):
    pid = pl.program_id(0)
    nbase = (pid // BLK_PER_IMG) * (NH * LQ)
    q = q_ref[...]
    v = jnp.dot(q, wv_ref[...], preferred_element_type=jnp.float32) + bv_ref[0:1, :]
    vbf_ref[...] = v.astype(jnp.bfloat16)
    sx = jnp.dot(v, wox_ref[...], preferred_element_type=jnp.float32) + box_ref[0:1, :]
    sy = jnp.dot(v, woy_ref[...], preferred_element_type=jnp.float32) + boy_ref[0:1, :]
    al = jnp.dot(v, wat_ref[...], preferred_element_type=jnp.float32) + bat_ref[0:1, :]
    al = al - jnp.max(al, axis=1, keepdims=True)
    e = jnp.exp(jnp.maximum(al, -75.0))
    s = jnp.dot(e, g_ref[...], preferred_element_type=jnp.float32)
    aw = e / s
    wlf = lcf_ref[0:1, :]
    hlf = lcf_ref[1:2, :]
    base = lci_ref[0:1, :] + nbase
    wli = lci_ref[1:2, :]
    x = rpx_ref[...] * wlf + sx - 0.5
    y = rpy_ref[...] * hlf + sy - 0.5
    x0 = jnp.floor(x)
    y0 = jnp.floor(y)
    fx1 = x - x0
    fx0 = 1.0 - fx1
    fy1 = y - y0
    fy0 = 1.0 - fy1
    idx_parts = []
    w_parts = []
    for dy in (0, 1):
        yi = y0 + dy
        vy = (yi >= 0.0) & (yi <= hlf - 1.0)
        yc = jnp.clip(yi, 0.0, hlf - 1.0).astype(jnp.int32)
        wy = fy1 if dy else fy0
        for dx in (0, 1):
            xi = x0 + dx
            vx = (xi >= 0.0) & (xi <= wlf - 1.0)
            xc = jnp.clip(xi, 0.0, wlf - 1.0).astype(jnp.int32)
            wx = fx1 if dx else fx0
            wc = aw * wy * wx * (vy & vx).astype(jnp.float32)
            idx_parts.append(base + yc * wli + xc)
            w_parts.append(wc)
    idx_ref[...] = jnp.concatenate(idx_parts, axis=1)
    w_ref[...] = jnp.concatenate(w_parts, axis=1)


def _row_spec(cols):
    return pl.BlockSpec((R, cols), lambda i: (i, 0))


def _full_spec(rows, cols):
    return pl.BlockSpec((rows, cols), lambda i: (0, 0))


_A_GRID = (NBLK,)
_A_IN_SPECS = [
    _row_spec(D),
    _row_spec(128),
    _row_spec(128),
    _full_spec(D, D),
    _full_spec(8, D),
    _full_spec(D, 128),
    _full_spec(8, 128),
    _full_spec(D, 128),
    _full_spec(8, 128),
    _full_spec(D, 128),
    _full_spec(8, 128),
    _full_spec(128, 128),
    _full_spec(8, 128),
    _full_spec(8, 128),
]
_A_OUT_SPECS = [_row_spec(D), _row_spec(512), _row_spec(512)]
_A_OUT_SHAPE = [
    jax.ShapeDtypeStruct((M, D), jnp.bfloat16),
    jax.ShapeDtypeStruct((M, 512), jnp.int32),
    jax.ShapeDtypeStruct((M, 512), jnp.float32),
]

_PERM = np.concatenate([
    h * DH + np.concatenate([np.arange(0, DH, 2), np.arange(1, DH, 2)])
    for h in range(NH)
])


def _mm_body(x_ref, w_ref, b_ref, o_ref):
    o_ref[...] = (jnp.dot(x_ref[...], w_ref[...],
                          preferred_element_type=jnp.float32) + b_ref[0:1, :])


_MM_IN_SPECS = [_row_spec(D), _full_spec(D, D), _full_spec(8, D)]
_MM_OUT_SPEC = _row_spec(D)


CH = 2
NCHK = PER_W // CH


def _sc_sample_body(vt_hbm, idx_hbm, w_hbm, out_hbm,
                    idxv0, idxv1, wv0, wv1, rows0, rows1, outv0, outv1,
                    semi0, semi1, semg0, semg1, semo0, semo1):
    wid = lax.axis_index("s") * 2 + lax.axis_index("c")
    m0 = wid * PER_W
    idxv = (idxv0, idxv1)
    wv = (wv0, wv1)
    rows = (rows0, rows1)
    outv = (outv0, outv1)
    semi = (semi0, semi1)
    semg = (semg0, semg1)
    semo = (semo0, semo1)

    def load_and_gather(g, b):
        mg = m0 + g * CH
        ci = pltpu.async_copy(idx_hbm.at[pl.ds(mg, CH)], idxv[b], semi[b])
        cw = pltpu.async_copy(w_hbm.at[pl.ds(mg, CH)], wv[b], semi[b])
        ci.wait()
        cw.wait()
        for mm in range(CH):
            pltpu.async_copy(vt_hbm.at[idxv[b].at[mm]],
                             rows[b].at[mm], semg[b])

    def wait_gathers(b):
        for mm in range(CH):
            pltpu.make_async_copy(vt_hbm.at[idxv[b].at[mm]],
                                  rows[b].at[mm], semg[b]).wait()

    def compute(g, b):
        for mm in range(CH):
            def body_h(h, c2, mm=mm):
                ae = [jnp.zeros((16,), jnp.float32) for _ in range(4)]
                ao = [jnp.zeros((16,), jnp.float32) for _ in range(4)]
                for c in range(4):
                    wrow = wv[b][mm, c, pl.ds(h * 16, 16)]
                    for j in range(16):
                        pos = c * 128 + h * 16 + j
                        wvec = jnp.full((16,), wrow[j], jnp.float32)
                        u = rows[b][mm, pos, :]
                        lo = lax.bitcast_convert_type(u << 16, jnp.float32)
                        hi = lax.bitcast_convert_type(u & jnp.int32(-65536),
                                                      jnp.float32)
                        ae[c] = ae[c] + lo * wvec
                        ao[c] = ao[c] + hi * wvec
                outv[b][mm, h, pl.ds(0, 16)] = (ae[0] + ae[1]) + (ae[2] + ae[3])
                outv[b][mm, h, pl.ds(16, 16)] = (ao[0] + ao[1]) + (ao[2] + ao[3])
                return c2

            lax.fori_loop(0, NH, body_h, 0)
        mg = m0 + g * CH
        pltpu.async_copy(outv[b], out_hbm.at[pl.ds(mg, CH)], semo[b])

    def wait_out(g, b):
        mg = m0 + g * CH
        pltpu.make_async_copy(outv[b], out_hbm.at[pl.ds(mg, CH)],
                              semo[b]).wait()

    load_and_gather(0, 0)

    def outer(go, carry):
        for b in (0, 1):
            g = go * 2 + b
            q = 1 - b

            @pl.when(g + 1 < NCHK)
            def _():
                load_and_gather(g + 1, q)

            wait_gathers(b)

            @pl.when(g >= 2)
            def _():
                wait_out(g - 2, b)

            compute(g, b)
        return carry

    lax.fori_loop(0, NCHK // 2, outer, 0)
    wait_out(NCHK - 2, 0)
    wait_out(NCHK - 1, 1)


@functools.lru_cache(maxsize=1)
def _get_sc_kernel():
    mesh = plsc.VectorSubcoreMesh(core_axis_name="c", subcore_axis_name="s")
    return pl.kernel(
        _sc_sample_body,
        out_type=jax.ShapeDtypeStruct((M, NH, DH), jnp.float32),
        mesh=mesh,
        compiler_params=pltpu.CompilerParams(use_tc_tiling_on_sc=False),
        scratch_types=[
            pltpu.VMEM((CH, 512), jnp.int32),
            pltpu.VMEM((CH, 512), jnp.int32),
            pltpu.VMEM((CH, 4, 128), jnp.float32),
            pltpu.VMEM((CH, 4, 128), jnp.float32),
            pltpu.VMEM((CH, 512, DH // 2), jnp.int32),
            pltpu.VMEM((CH, 512, DH // 2), jnp.int32),
            pltpu.VMEM((CH, NH, DH), jnp.float32),
            pltpu.VMEM((CH, NH, DH), jnp.float32),
            pltpu.SemaphoreType.DMA,
            pltpu.SemaphoreType.DMA,
            pltpu.SemaphoreType.DMA,
            pltpu.SemaphoreType.DMA,
            pltpu.SemaphoreType.DMA,
            pltpu.SemaphoreType.DMA,
        ],
    )


def _stage_a(q2, rpx, rpy, W_value, b_value, W_off, b_off, W_attn, b_attn):
    wv_t = W_value.T
    wox_t = W_off[0::2].T
    woy_t = W_off[1::2].T
    wat_t = W_attn.T
    bv8 = jnp.broadcast_to(b_value[None, :], (8, D))
    box8 = jnp.broadcast_to(b_off[0::2][None, :], (8, 128))
    boy8 = jnp.broadcast_to(b_off[1::2][None, :], (8, 128))
    bat8 = jnp.broadcast_to(b_attn[None, :], (8, 128))
    return pl.pallas_call(
        _stage_a_body,
        grid=_A_GRID,
        in_specs=_A_IN_SPECS,
        out_specs=_A_OUT_SPECS,
        out_shape=_A_OUT_SHAPE,
    )(q2, rpx, rpy, wv_t, bv8, wox_t, box8, woy_t, boy8, wat_t, bat8,
      jnp.asarray(_G), jnp.asarray(_LCF), jnp.asarray(_LCI))


def kernel(query, reference_points, input_spatial_shapes,
           input_level_start_index, temporal_points, temporal_points_weights,
           W_value, b_value, W_off, b_off, W_attn, b_attn, W_out, b_out):
    q2 = query.reshape(M, D)
    rp = reference_points.reshape(M, NL, 2)
    rpx = jnp.broadcast_to(rp[:, None, :, None, 0], (M, NH, NL, NP)).reshape(M, 128)
    rpy = jnp.broadcast_to(rp[:, None, :, None, 1], (M, NH, NL, NP)).reshape(M, 128)
    vbf, idx2, w2 = _stage_a(q2, rpx, rpy, W_value, b_value, W_off, b_off,
                             W_attn, b_attn)
    vt = vbf.reshape(NIMG, LQ, NH, DH).transpose(0, 2, 1, 3).reshape(TAB, DH)
    vt32 = lax.bitcast_convert_type(vt.reshape(TAB, DH // 2, 2), jnp.int32)
    sampled = _get_sc_kernel()(vt32, idx2,
                               w2.reshape(M, 4, 128))
    out2 = pl.pallas_call(
        _mm_body,
        grid=_A_GRID,
        in_specs=_MM_IN_SPECS,
        out_specs=_MM_OUT_SPEC,
        out_shape=jax.ShapeDtypeStruct((M, D), jnp.float32),
    )(sampled.reshape(M, D), W_out.T[_PERM],
      jnp.broadcast_to(b_out[None, :], (8, D)))
    return out2.reshape(NIMG, LQ, D)


# speedup vs baseline: 1.1343x; 1.1279x over previous
import functools

import jax
import jax.numpy as jnp
import numpy as np
from jax import lax
from jax.experimental import pallas as pl
from jax.experimental.pallas import tpu as pltpu
from jax.experimental.pallas import tpu_sc as plsc

D = 256
NH = 8
NL = 4
NP = 4
DH = D // NH
SPAT = [(64, 64), (32, 32), (16, 16), (8, 8)]
LQ = sum(h * w for h, w in SPAT)
NIMG = 4
M = NIMG * LQ
TAB = NIMG * NH * LQ
R = 680
NBLK = M // R
BLK_PER_IMG = LQ // R
NWORK = 32
PER_W = M // NWORK

_LVL_START = np.array([0, 4096, 5120, 5376], dtype=np.int64)

_wl_lane = np.tile(np.repeat(np.array([w for _, w in SPAT], np.float32), NP), NH)
_hl_lane = np.tile(np.repeat(np.array([h for h, _ in SPAT], np.float32), NP), NH)
_base_lane = (np.repeat(np.arange(NH, dtype=np.int64) * LQ, NL * NP)
              + np.tile(np.repeat(_LVL_START, NP), NH)).astype(np.int32)
_wli_lane = _wl_lane.astype(np.int32)

_LCF = np.zeros((8, 128), np.float32)
_LCF[0] = _wl_lane
_LCF[1] = _hl_lane
_LCI = np.zeros((8, 128), np.int32)
_LCI[0] = _base_lane
_LCI[1] = _wli_lane
_G = np.kron(np.eye(NH, dtype=np.float32), np.ones((16, 16), np.float32))


def _stage_a_body(q_ref, rpx_ref, rpy_ref, wv_ref, bv_ref, wox_ref, box_ref,
                  woy_ref, boy_ref, wat_ref, bat_ref, g_ref, lcf_ref, lci_ref,
                  v_ref, idx_ref, w_ref):
    pid = pl.program_id(0)
    nbase = (pid // BLK_PER_IMG) * (NH * LQ)
    q = q_ref[...]
    v = jnp.dot(q, wv_ref[...], preferred_element_type=jnp.float32) + bv_ref[0:1, :]
    v_ref[...] = v
    sx = jnp.dot(v, wox_ref[...], preferred_element_type=jnp.float32) + box_ref[0:1, :]
    sy = jnp.dot(v, woy_ref[...], preferred_element_type=jnp.float32) + boy_ref[0:1, :]
    al = jnp.dot(v, wat_ref[...], preferred_element_type=jnp.float32) + bat_ref[0:1, :]
    al = al - jnp.max(al, axis=1, keepdims=True)
    e = jnp.exp(jnp.maximum(al, -75.0))
    s = jnp.dot(e, g_ref[...], preferred_element_type=jnp.float32)
    aw = e / s
    wlf = lcf_ref[0:1, :]
    hlf = lcf_ref[1:2, :]
    base = lci_ref[0:1, :] + nbase
    wli = lci_ref[1:2, :]
    x = rpx_ref[...] * wlf + sx - 0.5
    y = rpy_ref[...] * hlf + sy - 0.5
    x0 = jnp.floor(x)
    y0 = jnp.floor(y)
    fx1 = x - x0
    fx0 = 1.0 - fx1
    fy1 = y - y0
    fy0 = 1.0 - fy1
    idx_parts = []
    w_parts = []
    for dy in (0, 1):
        yi = y0 + dy
        vy = (yi >= 0.0) & (yi <= hlf - 1.0)
        yc = jnp.clip(yi, 0.0, hlf - 1.0).astype(jnp.int32)
        wy = fy1 if dy else fy0
        for dx in (0, 1):
            xi = x0 + dx
            vx = (xi >= 0.0) & (xi <= wlf - 1.0)
            xc = jnp.clip(xi, 0.0, wlf - 1.0).astype(jnp.int32)
            wx = fx1 if dx else fx0
            wc = aw * wy * wx * (vy & vx).astype(jnp.float32)
            idx_parts.append(base + yc * wli + xc)
            w_parts.append(wc)
    idx_ref[...] = jnp.concatenate(idx_parts, axis=1)
    w_ref[...] = jnp.concatenate(w_parts, axis=1)


def _row_spec(cols):
    return pl.BlockSpec((R, cols), lambda i: (i, 0))


def _full_spec(rows, cols):
    return pl.BlockSpec((rows, cols), lambda i: (0, 0))


_A_GRID = (NBLK,)
_A_IN_SPECS = [
    _row_spec(D),
    _row_spec(128),
    _row_spec(128),
    _full_spec(D, D),
    _full_spec(8, D),
    _full_spec(D, 128),
    _full_spec(8, 128),
    _full_spec(D, 128),
    _full_spec(8, 128),
    _full_spec(D, 128),
    _full_spec(8, 128),
    _full_spec(128, 128),
    _full_spec(8, 128),
    _full_spec(8, 128),
]
_A_OUT_SPECS = [_row_spec(D), _row_spec(512), _row_spec(512)]
_A_OUT_SHAPE = [
    jax.ShapeDtypeStruct((M, D), jnp.float32),
    jax.ShapeDtypeStruct((M, 512), jnp.int32),
    jax.ShapeDtypeStruct((M, 512), jnp.float32),
]


def _mm_body(x_ref, w_ref, b_ref, o_ref):
    o_ref[...] = (jnp.dot(x_ref[...], w_ref[...],
                          preferred_element_type=jnp.float32) + b_ref[0:1, :])


_MM_IN_SPECS = [_row_spec(D), _full_spec(D, D), _full_spec(8, D)]
_MM_OUT_SPEC = _row_spec(D)


CH = 2
NCHK = PER_W // CH


def _sc_sample_body(vt_hbm, idx_hbm, w_hbm, out_hbm,
                    idxv0, idxv1, idxv2, idxv3, wv0, wv1, wv2, wv3,
                    rows0, rows1, outv0, outv1,
                    semi0, semi1, semi2, semi3, semg0, semg1, semo0, semo1):
    wid = lax.axis_index("s") * 2 + lax.axis_index("c")
    m0 = wid * PER_W
    idxv = (idxv0, idxv1, idxv2, idxv3)
    wv = (wv0, wv1, wv2, wv3)
    rows = (rows0, rows1)
    outv = (outv0, outv1)
    semi = (semi0, semi1, semi2, semi3)
    semg = (semg0, semg1)
    semo = (semo0, semo1)

    def issue_in(g, r):
        mg = m0 + g * CH
        pltpu.async_copy(idx_hbm.at[pl.ds(mg, CH)], idxv[r], semi[r])
        pltpu.async_copy(w_hbm.at[pl.ds(mg, CH)], wv[r], semi[r])

    def wait_in(g, r):
        mg = m0 + g * CH
        pltpu.make_async_copy(idx_hbm.at[pl.ds(mg, CH)], idxv[r],
                              semi[r]).wait()
        pltpu.make_async_copy(w_hbm.at[pl.ds(mg, CH)], wv[r], semi[r]).wait()

    def fire_gathers(r, b):
        for mm in range(CH):
            for c in range(4):
                pltpu.async_copy(vt_hbm.at[idxv[r].at[mm, c]],
                                 rows[b].at[mm, c], semg[b])

    def wait_gathers(r, b):
        for mm in range(CH):
            for c in range(4):
                pltpu.make_async_copy(vt_hbm.at[idxv[r].at[mm, c]],
                                      rows[b].at[mm, c], semg[b]).wait()

    def compute(g, r, b):
        for mm in range(CH):
            def body_h(h, c2, mm=mm):
                a0 = [jnp.zeros((16,), jnp.float32) for _ in range(4)]
                a1 = [jnp.zeros((16,), jnp.float32) for _ in range(4)]
                for c in range(4):
                    wrow = wv[r][mm, c, pl.ds(h * 16, 16)]
                    for j in range(16):
                        pos = h * 16 + j
                        wvec = jnp.full((16,), wrow[j], jnp.float32)
                        a0[c] = a0[c] + rows[b][mm, c, pos, pl.ds(0, 16)] * wvec
                        a1[c] = a1[c] + rows[b][mm, c, pos, pl.ds(16, 16)] * wvec
                outv[b][mm, h, pl.ds(0, 16)] = (a0[0] + a0[1]) + (a0[2] + a0[3])
                outv[b][mm, h, pl.ds(16, 16)] = (a1[0] + a1[1]) + (a1[2] + a1[3])
                return c2

            lax.fori_loop(0, NH, body_h, 0)
        mg = m0 + g * CH
        pltpu.async_copy(outv[b], out_hbm.at[pl.ds(mg, CH)], semo[b])

    def wait_out(g, b):
        mg = m0 + g * CH
        pltpu.make_async_copy(outv[b], out_hbm.at[pl.ds(mg, CH)],
                              semo[b]).wait()

    issue_in(0, 0)
    issue_in(1, 1)
    issue_in(2, 2)
    wait_in(0, 0)
    fire_gathers(0, 0)

    def outer(go, carry):
        for gg in range(4):
            g = go * 4 + gg
            b = gg % 2
            r = gg

            @pl.when(g + 1 < NCHK)
            def _(g=g, gg=gg, b=b):
                wait_in(g + 1, (gg + 1) % 4)
                fire_gathers((gg + 1) % 4, 1 - b)

            wait_gathers(r, b)

            @pl.when(g + 3 < NCHK)
            def _(g=g, gg=gg):
                issue_in(g + 3, (gg + 3) % 4)

            @pl.when(g >= 2)
            def _(g=g, b=b):
                wait_out(g - 2, b)

            compute(g, r, b)
        return carry

    lax.fori_loop(0, NCHK // 4, outer, 0)
    wait_out(NCHK - 2, 0)
    wait_out(NCHK - 1, 1)


@functools.lru_cache(maxsize=1)
def _get_sc_kernel():
    mesh = plsc.VectorSubcoreMesh(core_axis_name="c", subcore_axis_name="s")
    return pl.kernel(
        _sc_sample_body,
        out_type=jax.ShapeDtypeStruct((M, NH, DH), jnp.float32),
        mesh=mesh,
        compiler_params=pltpu.CompilerParams(use_tc_tiling_on_sc=False),
        scratch_types=[
            pltpu.VMEM((CH, 4, 128), jnp.int32),
            pltpu.VMEM((CH, 4, 128), jnp.int32),
            pltpu.VMEM((CH, 4, 128), jnp.int32),
            pltpu.VMEM((CH, 4, 128), jnp.int32),
            pltpu.VMEM((CH, 4, 128), jnp.float32),
            pltpu.VMEM((CH, 4, 128), jnp.float32),
            pltpu.VMEM((CH, 4, 128), jnp.float32),
            pltpu.VMEM((CH, 4, 128), jnp.float32),
            pltpu.VMEM((CH, 4, 128, DH), jnp.float32),
            pltpu.VMEM((CH, 4, 128, DH), jnp.float32),
            pltpu.VMEM((CH, NH, DH), jnp.float32),
            pltpu.VMEM((CH, NH, DH), jnp.float32),
            pltpu.SemaphoreType.DMA,
            pltpu.SemaphoreType.DMA,
            pltpu.SemaphoreType.DMA,
            pltpu.SemaphoreType.DMA,
            pltpu.SemaphoreType.DMA,
            pltpu.SemaphoreType.DMA,
            pltpu.SemaphoreType.DMA,
            pltpu.SemaphoreType.DMA,
        ],
    )


def _stage_a(q2, rpx, rpy, W_value, b_value, W_off, b_off, W_attn, b_attn):
    wv_t = W_value.T
    wox_t = W_off[0::2].T
    woy_t = W_off[1::2].T
    wat_t = W_attn.T
    bv8 = jnp.broadcast_to(b_value[None, :], (8, D))
    box8 = jnp.broadcast_to(b_off[0::2][None, :], (8, 128))
    boy8 = jnp.broadcast_to(b_off[1::2][None, :], (8, 128))
    bat8 = jnp.broadcast_to(b_attn[None, :], (8, 128))
    return pl.pallas_call(
        _stage_a_body,
        grid=_A_GRID,
        in_specs=_A_IN_SPECS,
        out_specs=_A_OUT_SPECS,
        out_shape=_A_OUT_SHAPE,
    )(q2, rpx, rpy, wv_t, bv8, wox_t, box8, woy_t, boy8, wat_t, bat8,
      jnp.asarray(_G), jnp.asarray(_LCF), jnp.asarray(_LCI))


def kernel(query, reference_points, input_spatial_shapes,
           input_level_start_index, temporal_points, temporal_points_weights,
           W_value, b_value, W_off, b_off, W_attn, b_attn, W_out, b_out):
    q2 = query.reshape(M, D)
    rp = reference_points.reshape(M, NL, 2)
    rpx = jnp.broadcast_to(rp[:, None, :, None, 0], (M, NH, NL, NP)).reshape(M, 128)
    rpy = jnp.broadcast_to(rp[:, None, :, None, 1], (M, NH, NL, NP)).reshape(M, 128)
    v2, idx2, w2 = _stage_a(q2, rpx, rpy, W_value, b_value, W_off, b_off,
                            W_attn, b_attn)
    vt = v2.reshape(NIMG, LQ, NH, DH).transpose(0, 2, 1, 3).reshape(TAB, DH)
    sampled = _get_sc_kernel()(vt, idx2.reshape(M, 4, 128),
                               w2.reshape(M, 4, 128))
    out2 = pl.pallas_call(
        _mm_body,
        grid=_A_GRID,
        in_specs=_MM_IN_SPECS,
        out_specs=_MM_OUT_SPEC,
        out_shape=jax.ShapeDtypeStruct((M, D), jnp.float32),
    )(sampled.reshape(M, D), W_out.T,
      jnp.broadcast_to(b_out[None, :], (8, D)))
    return out2.reshape(NIMG, LQ, D)
